# baseline (speedup 1.0000x reference)
import functools

import jax
import jax.numpy as jnp
import numpy as np
from jax import lax
from jax.experimental import pallas as pl
from jax.experimental.pallas import tpu as pltpu
from jax.experimental.pallas import tpu_sc as plsc

N = 10000
E = 320000
R = 8
RN = R * N
BN_EPS = 1e-5

NC = 2
NS = 16
NW = NC * NS
DC = 16

EPT = E // NW
EC = 125
GC = EPT // EC
AROWS = RN // NS
ZROWS = 625
FROWS = N // NS

SL = 4

TN = 1000
NT = N // TN

@functools.lru_cache(maxsize=1)
def _mesh():
    return plsc.VectorSubcoreMesh(
        core_axis_name="c", subcore_axis_name="s", num_cores=NC, num_subcores=NS)


def _sc_count(segT):
    @functools.partial(
        pl.kernel,
        out_type=jax.ShapeDtypeStruct((NC, RN, DC), jnp.float32),
        mesh=_mesh(),
        compiler_params=pltpu.CompilerParams(use_tc_tiling_on_sc=False),
        scratch_types=[
            pltpu.VMEM_SHARED((RN, DC), jnp.float32),
            pltpu.VMEM((GC, EC), jnp.int32),
            pltpu.VMEM((EC, DC), jnp.float32),
            pltpu.VMEM((ZROWS, DC), jnp.float32),
        ],
    )
    def k(seg_hbm, out_hbm, cnt_sh, seg_v, ones_v, zbuf):
        c = lax.axis_index("c")
        s = lax.axis_index("s")
        wid = c * NS + s
        pltpu.sync_copy(seg_hbm.at[wid], seg_v)

        def fill_z(i, carry):
            zbuf[i, :] = jnp.zeros((DC,), jnp.float32)
            return carry

        lax.fori_loop(0, ZROWS, fill_z, 0)

        def fill_o(i, carry):
            ones_v[i, :] = jnp.ones((DC,), jnp.float32)
            return carry

        lax.fori_loop(0, EC, fill_o, 0)

        for z in range(AROWS // ZROWS):
            pltpu.sync_copy(zbuf, cnt_sh.at[pl.ds(s * AROWS + z * ZROWS, ZROWS)])
        plsc.subcore_barrier()

        def edge_step(j, carry):
            pltpu.sync_copy(ones_v, cnt_sh.at[seg_v.at[j]], add=True)
            return carry

        lax.fori_loop(0, GC, edge_step, 0)
        plsc.subcore_barrier()
        pltpu.sync_copy(
            cnt_sh.at[pl.ds(s * AROWS, AROWS)],
            out_hbm.at[c, pl.ds(s * AROWS, AROWS), :],
        )

    return k(segT)


def _sc_agg(f, srcT, segT):
    D = f.shape[1]
    C = D // DC

    @functools.partial(
        pl.kernel,
        out_type=jax.ShapeDtypeStruct((NC, RN, D), jnp.float32),
        mesh=_mesh(),
        compiler_params=pltpu.CompilerParams(use_tc_tiling_on_sc=False),
        scratch_types=[
            pltpu.VMEM_SHARED((N, DC), jnp.float32),
            pltpu.VMEM_SHARED((RN, DC), jnp.float32),
            pltpu.VMEM((GC, EC), jnp.int32),
            pltpu.VMEM((GC, EC), jnp.int32),
            [pltpu.VMEM((EC, DC), jnp.float32) for _ in range(SL)],
            pltpu.VMEM((ZROWS, DC), jnp.float32),
            [pltpu.SemaphoreType.DMA for _ in range(SL)],
            [pltpu.SemaphoreType.DMA for _ in range(SL)],
        ],
    )
    def k(f_hbm, src_hbm, seg_hbm, out_hbm,
          fch_sh, agg_sh, src_v, seg_v, gbufs, zbuf, gsems, ssems):
        c = lax.axis_index("c")
        s = lax.axis_index("s")
        wid = c * NS + s
        pltpu.sync_copy(src_hbm.at[wid], src_v)
        pltpu.sync_copy(seg_hbm.at[wid], seg_v)

        def fill_z(i, carry):
            zbuf[i, :] = jnp.zeros((DC,), jnp.float32)
            return carry

        lax.fori_loop(0, ZROWS, fill_z, 0)

        for ci in range(C):
            col = ci * DC
            for z in range(AROWS // ZROWS):
                pltpu.sync_copy(zbuf, agg_sh.at[pl.ds(s * AROWS + z * ZROWS, ZROWS)])
            pltpu.sync_copy(
                f_hbm.at[pl.ds(s * FROWS, FROWS), pl.ds(col, DC)],
                fch_sh.at[pl.ds(s * FROWS, FROWS)],
            )
            plsc.subcore_barrier()

            def edge_step(sp, carry):
                gds = []
                for t in range(SL):
                    @pl.when(sp > 0)
                    def _drain():
                        pltpu.make_async_copy(
                            gbufs[t], agg_sh.at[seg_v.at[0]], ssems[t]).wait()
                    gds.append(pltpu.async_copy(
                        fch_sh.at[src_v.at[sp * SL + t]], gbufs[t], gsems[t]))
                for t in range(SL):
                    gds[t].wait()
                    pltpu.async_copy(
                        gbufs[t], agg_sh.at[seg_v.at[sp * SL + t]],
                        ssems[t], add=True)
                return carry

            lax.fori_loop(0, GC // SL, edge_step, 0)
            for t in range(SL):
                pltpu.make_async_copy(
                    gbufs[t], agg_sh.at[seg_v.at[0]], ssems[t]).wait()
            plsc.subcore_barrier()
            pltpu.sync_copy(
                agg_sh.at[pl.ds(s * AROWS, AROWS)],
                out_hbm.at[c, pl.ds(s * AROWS, AROWS), pl.ds(col, DC)],
            )

    return k(f, srcT, segT)


def _block_call(feats, aggs, cnt2, bp, *, residual_idx, epilogue, epi_ws):
    P = len(feats)
    dims = [f.shape[1] for f in feats]
    offs = np.concatenate([[0], np.cumsum(dims)]).tolist()
    dout = bp["Wroot"].shape[1]
    inv_bn = float(1.0 / np.sqrt(1.0 + BN_EPS))

    def body(*refs):
        it = iter(refs)
        f_refs = [next(it) for _ in range(P)]
        a_refs = [next(it) for _ in range(P)]
        cnt_ref, wroot, bias, wrel, gamma, beta, pa = (next(it) for _ in range(7))
        e_refs = [next(it) for _ in range(len(epi_ws))]
        if epilogue == "latent":
            o_refs = [next(it), next(it)]
        else:
            o_refs = [next(it)]
        acc = next(it)

        r = pl.program_id(1)

        @pl.when(r == 0)
        def _init():
            a = bias[...]
            for p in range(P):
                a = a + jnp.dot(f_refs[p][...], wroot[offs[p]:offs[p + 1], :],
                                preferred_element_type=jnp.float32)
            acc[...] = a

        @pl.when(r > 0)
        def _accum():
            cnt = cnt_ref[0, :, 0:1] + cnt_ref[1, :, 0:1]
            scale = 1.0 / jnp.maximum(cnt, 1.0)
            a = acc[...]
            for p in range(P):
                m = (a_refs[p][0] + a_refs[p][1]) * scale
                a = a + jnp.dot(m, wrel[0, offs[p]:offs[p + 1], :],
                                preferred_element_type=jnp.float32)
            acc[...] = a

        @pl.when(r == R)
        def _epilogue():
            h = acc[...] * (inv_bn * gamma[...]) + beta[...]
            al = pa[0, 0]
            h = jnp.maximum(h, 0.0) + al * jnp.minimum(h, 0.0)
            if residual_idx is not None:
                h = h + f_refs[residual_idx][...]
            if epilogue == "plain":
                o_refs[0][...] = h
            elif epilogue == "latent":
                o_refs[0][...] = jnp.dot(h, e_refs[0][...],
                                         preferred_element_type=jnp.float32) + e_refs[1][...]
                o_refs[1][...] = jnp.dot(h, e_refs[2][...],
                                         preferred_element_type=jnp.float32) + e_refs[3][...]
            else:
                o_refs[0][...] = jnp.dot(h, e_refs[0][...],
                                         preferred_element_type=jnp.float32) + e_refs[1][...]

    def rm1(r):
        return jnp.maximum(r, 1) - 1

    in_specs = []
    for p in range(P):
        in_specs.append(pl.BlockSpec((TN, dims[p]), lambda i, r: (i, 0)))
    for p in range(P):
        in_specs.append(pl.BlockSpec(
            (NC, TN, dims[p]), lambda i, r: (0, rm1(r) * NT + i, 0)))
    in_specs.append(pl.BlockSpec((NC, TN, DC), lambda i, r: (0, rm1(r) * NT + i, 0)))
    in_specs.append(pl.BlockSpec((sum(dims), dout), lambda i, r: (0, 0)))
    in_specs.append(pl.BlockSpec((1, dout), lambda i, r: (0, 0)))
    in_specs.append(pl.BlockSpec((1, sum(dims), dout), lambda i, r: (rm1(r), 0, 0)))
    in_specs.append(pl.BlockSpec((1, dout), lambda i, r: (0, 0)))
    in_specs.append(pl.BlockSpec((1, dout), lambda i, r: (0, 0)))
    in_specs.append(pl.BlockSpec((1, 1), lambda i, r: (0, 0)))
    for w in epi_ws:
        in_specs.append(pl.BlockSpec(w.shape, lambda i, r: (0,) * w.ndim))

    if epilogue == "latent":
        lat = epi_ws[0].shape[1]
        out_shape = [jax.ShapeDtypeStruct((N, lat), jnp.float32),
                     jax.ShapeDtypeStruct((N, lat), jnp.float32)]
        out_specs = [pl.BlockSpec((TN, lat), lambda i, r: (i, 0)),
                     pl.BlockSpec((TN, lat), lambda i, r: (i, 0))]
    else:
        od = epi_ws[0].shape[1] if epilogue == "proj" else dout
        out_shape = [jax.ShapeDtypeStruct((N, od), jnp.float32)]
        out_specs = [pl.BlockSpec((TN, od), lambda i, r: (i, 0))]

    wrel3 = bp["Wrel"]
    args = (list(feats) + list(aggs)
            + [cnt2, bp["Wroot"], bp["bias"].reshape(1, dout), wrel3,
               bp["gamma"].reshape(1, dout), bp["beta"].reshape(1, dout),
               bp["prelu_a"].reshape(1, 1)]
            + list(epi_ws))

    outs = pl.pallas_call(
        body,
        grid=(NT, R + 1),
        in_specs=in_specs,
        out_specs=out_specs,
        out_shape=out_shape,
        scratch_shapes=[pltpu.VMEM((TN, dout), jnp.float32)],
    )(*args)
    return outs


def kernel(x, edge_index, edge_attr, params):
    src = edge_index[0]
    dst = edge_index[1]
    etype = edge_attr[:, 4].astype(jnp.int32)
    seg = etype * N + dst
    srcT = src.reshape(NW, GC, EC)
    segT = seg.reshape(NW, GC, EC)

    cnt2 = _sc_count(segT)
    aggx = _sc_agg(x, srcT, segT)
    (h0,) = _block_call([x], [aggx], cnt2, params["enc0"],
                        residual_idx=None, epilogue="plain", epi_ws=[])
    aggh = _sc_agg(h0, srcT, segT)
    mu, logvar = _block_call(
        [h0], [aggh], cnt2, params["enc1"], residual_idx=0, epilogue="latent",
        epi_ws=[params["W_mu"], params["b_mu"].reshape(1, -1),
                params["W_lv"], params["b_lv"].reshape(1, -1)])
    aggz = _sc_agg(mu, srcT, segT)
    (d0,) = _block_call([mu, x], [aggz, aggx], cnt2, params["dec0"],
                        residual_idx=None, epilogue="plain", epi_ws=[])
    aggd = _sc_agg(d0, srcT, segT)
    (out,) = _block_call(
        [d0], [aggd], cnt2, params["dec1"], residual_idx=0, epilogue="proj",
        epi_ws=[params["W_out"], params["b_out"].reshape(1, -1)])
    return out, mu, logvar


# speedup vs baseline: 1.2982x; 1.2982x over previous
import functools

import jax
import jax.numpy as jnp
import numpy as np
from jax import lax
from jax.experimental import pallas as pl
from jax.experimental.pallas import tpu as pltpu
from jax.experimental.pallas import tpu_sc as plsc

N = 10000
E = 320000
R = 8
RN = R * N
BN_EPS = 1e-5

NC = 2
NS = 16
NW = NC * NS
DC = 16

EPT = E // NW
EC = 125
GC = EPT // EC
AROWS = RN // NS
ZROWS = 625
FROWS = N // NS

SL = 4

TN = 1000
NT = N // TN

@functools.lru_cache(maxsize=1)
def _mesh():
    return plsc.VectorSubcoreMesh(
        core_axis_name="c", subcore_axis_name="s", num_cores=NC, num_subcores=NS)


def _sc_count(segT):
    @functools.partial(
        pl.kernel,
        out_type=jax.ShapeDtypeStruct((NC, RN, DC), jnp.float32),
        mesh=_mesh(),
        compiler_params=pltpu.CompilerParams(use_tc_tiling_on_sc=False),
        scratch_types=[
            pltpu.VMEM_SHARED((RN, DC), jnp.float32),
            pltpu.VMEM((GC, EC), jnp.int32),
            pltpu.VMEM((EC, DC), jnp.float32),
            pltpu.VMEM((ZROWS, DC), jnp.float32),
        ],
    )
    def k(seg_hbm, out_hbm, cnt_sh, seg_v, ones_v, zbuf):
        c = lax.axis_index("c")
        s = lax.axis_index("s")
        pltpu.sync_copy(seg_hbm.at[s, pl.ds(c * GC, GC)], seg_v)

        def fill_z(i, carry):
            zbuf[i, :] = jnp.zeros((DC,), jnp.float32)
            return carry

        lax.fori_loop(0, ZROWS, fill_z, 0)

        def fill_o(i, carry):
            ones_v[i, :] = jnp.ones((DC,), jnp.float32)
            return carry

        lax.fori_loop(0, EC, fill_o, 0)

        for z in range(AROWS // ZROWS):
            pltpu.sync_copy(zbuf, cnt_sh.at[pl.ds(s * AROWS + z * ZROWS, ZROWS)])
        plsc.subcore_barrier()

        def edge_step(j, carry):
            pltpu.sync_copy(ones_v, cnt_sh.at[seg_v.at[j]], add=True)
            return carry

        lax.fori_loop(0, GC, edge_step, 0)
        plsc.subcore_barrier()
        pltpu.sync_copy(
            cnt_sh.at[pl.ds(s * AROWS, AROWS)],
            out_hbm.at[c, pl.ds(s * AROWS, AROWS), :],
        )

    return k(segT)


def _sc_agg(f, srcT, segT):
    D = f.shape[1]
    C = D // DC
    CH = C // NC

    @functools.partial(
        pl.kernel,
        out_type=jax.ShapeDtypeStruct((RN, D), jnp.float32),
        mesh=_mesh(),
        compiler_params=pltpu.CompilerParams(use_tc_tiling_on_sc=False),
        scratch_types=[
            pltpu.VMEM_SHARED((N, DC), jnp.float32),
            pltpu.VMEM_SHARED((RN, DC), jnp.float32),
            pltpu.VMEM((GC, EC), jnp.int32),
            pltpu.VMEM((GC, EC), jnp.int32),
            [pltpu.VMEM((EC, DC), jnp.float32) for _ in range(SL)],
            pltpu.VMEM((ZROWS, DC), jnp.float32),
            [pltpu.SemaphoreType.DMA for _ in range(SL)],
            [pltpu.SemaphoreType.DMA for _ in range(SL)],
        ],
    )
    def k(f_hbm, src_hbm, seg_hbm, out_hbm,
          fch_sh, agg_sh, src_v, seg_v, gbufs, zbuf, gsems, ssems):
        c = lax.axis_index("c")
        s = lax.axis_index("s")

        def fill_z(i, carry):
            zbuf[i, :] = jnp.zeros((DC,), jnp.float32)
            return carry

        lax.fori_loop(0, ZROWS, fill_z, 0)

        for ci in range(CH):
            col = (c * CH + ci) * DC
            for z in range(AROWS // ZROWS):
                pltpu.sync_copy(zbuf, agg_sh.at[pl.ds(s * AROWS + z * ZROWS, ZROWS)])
            pltpu.sync_copy(
                f_hbm.at[pl.ds(s * FROWS, FROWS), pl.ds(col, DC)],
                fch_sh.at[pl.ds(s * FROWS, FROWS)],
            )
            plsc.subcore_barrier()

            for w in range(NC):
                pltpu.sync_copy(src_hbm.at[s, pl.ds(w * GC, GC)], src_v)
                pltpu.sync_copy(seg_hbm.at[s, pl.ds(w * GC, GC)], seg_v)

                def edge_step(sp, carry):
                    gds = [
                        pltpu.async_copy(
                            fch_sh.at[src_v.at[sp * SL + t]], gbufs[t], gsems[t])
                        for t in range(SL)
                    ]
                    sds = []
                    for t in range(SL):
                        gds[t].wait()
                        sds.append(pltpu.async_copy(
                            gbufs[t], agg_sh.at[seg_v.at[sp * SL + t]],
                            ssems[t], add=True))
                    for t in range(SL):
                        sds[t].wait()
                    return carry

                lax.fori_loop(0, GC // SL, edge_step, 0)
            plsc.subcore_barrier()
            pltpu.sync_copy(
                agg_sh.at[pl.ds(s * AROWS, AROWS)],
                out_hbm.at[pl.ds(s * AROWS, AROWS), pl.ds(col, DC)],
            )

    return k(f, srcT, segT)


def _block_call(feats, aggs, cnt2, bp, *, residual_idx, epilogue, epi_ws):
    P = len(feats)
    dims = [f.shape[1] for f in feats]
    offs = np.concatenate([[0], np.cumsum(dims)]).tolist()
    dout = bp["Wroot"].shape[1]
    inv_bn = float(1.0 / np.sqrt(1.0 + BN_EPS))

    def body(*refs):
        it = iter(refs)
        f_refs = [next(it) for _ in range(P)]
        a_refs = [next(it) for _ in range(P)]
        cnt_ref, wroot, bias, wrel, gamma, beta, pa = (next(it) for _ in range(7))
        e_refs = [next(it) for _ in range(len(epi_ws))]
        if epilogue == "latent":
            o_refs = [next(it), next(it)]
        else:
            o_refs = [next(it)]
        acc = next(it)

        r = pl.program_id(1)

        @pl.when(r == 0)
        def _init():
            a = bias[...]
            for p in range(P):
                a = a + jnp.dot(f_refs[p][...], wroot[offs[p]:offs[p + 1], :],
                                preferred_element_type=jnp.float32)
            acc[...] = a

        @pl.when(r > 0)
        def _accum():
            cnt = cnt_ref[0, :, 0:1] + cnt_ref[1, :, 0:1]
            scale = 1.0 / jnp.maximum(cnt, 1.0)
            a = acc[...]
            for p in range(P):
                m = a_refs[p][...] * scale
                a = a + jnp.dot(m, wrel[0, offs[p]:offs[p + 1], :],
                                preferred_element_type=jnp.float32)
            acc[...] = a

        @pl.when(r == R)
        def _epilogue():
            h = acc[...] * (inv_bn * gamma[...]) + beta[...]
            al = pa[0, 0]
            h = jnp.maximum(h, 0.0) + al * jnp.minimum(h, 0.0)
            if residual_idx is not None:
                h = h + f_refs[residual_idx][...]
            if epilogue == "plain":
                o_refs[0][...] = h
            elif epilogue == "latent":
                o_refs[0][...] = jnp.dot(h, e_refs[0][...],
                                         preferred_element_type=jnp.float32) + e_refs[1][...]
                o_refs[1][...] = jnp.dot(h, e_refs[2][...],
                                         preferred_element_type=jnp.float32) + e_refs[3][...]
            else:
                o_refs[0][...] = jnp.dot(h, e_refs[0][...],
                                         preferred_element_type=jnp.float32) + e_refs[1][...]

    def rm1(r):
        return jnp.maximum(r, 1) - 1

    in_specs = []
    for p in range(P):
        in_specs.append(pl.BlockSpec((TN, dims[p]), lambda i, r: (i, 0)))
    for p in range(P):
        in_specs.append(pl.BlockSpec(
            (TN, dims[p]), lambda i, r: (rm1(r) * NT + i, 0)))
    in_specs.append(pl.BlockSpec((NC, TN, DC), lambda i, r: (0, rm1(r) * NT + i, 0)))
    in_specs.append(pl.BlockSpec((sum(dims), dout), lambda i, r: (0, 0)))
    in_specs.append(pl.BlockSpec((1, dout), lambda i, r: (0, 0)))
    in_specs.append(pl.BlockSpec((1, sum(dims), dout), lambda i, r: (rm1(r), 0, 0)))
    in_specs.append(pl.BlockSpec((1, dout), lambda i, r: (0, 0)))
    in_specs.append(pl.BlockSpec((1, dout), lambda i, r: (0, 0)))
    in_specs.append(pl.BlockSpec((1, 1), lambda i, r: (0, 0)))
    for w in epi_ws:
        in_specs.append(pl.BlockSpec(w.shape, lambda i, r: (0,) * w.ndim))

    if epilogue == "latent":
        lat = epi_ws[0].shape[1]
        out_shape = [jax.ShapeDtypeStruct((N, lat), jnp.float32),
                     jax.ShapeDtypeStruct((N, lat), jnp.float32)]
        out_specs = [pl.BlockSpec((TN, lat), lambda i, r: (i, 0)),
                     pl.BlockSpec((TN, lat), lambda i, r: (i, 0))]
    else:
        od = epi_ws[0].shape[1] if epilogue == "proj" else dout
        out_shape = [jax.ShapeDtypeStruct((N, od), jnp.float32)]
        out_specs = [pl.BlockSpec((TN, od), lambda i, r: (i, 0))]

    wrel3 = bp["Wrel"]
    args = (list(feats) + list(aggs)
            + [cnt2, bp["Wroot"], bp["bias"].reshape(1, dout), wrel3,
               bp["gamma"].reshape(1, dout), bp["beta"].reshape(1, dout),
               bp["prelu_a"].reshape(1, 1)]
            + list(epi_ws))

    outs = pl.pallas_call(
        body,
        grid=(NT, R + 1),
        in_specs=in_specs,
        out_specs=out_specs,
        out_shape=out_shape,
        scratch_shapes=[pltpu.VMEM((TN, dout), jnp.float32)],
    )(*args)
    return outs


def kernel(x, edge_index, edge_attr, params):
    src = edge_index[0]
    dst = edge_index[1]
    etype = edge_attr[:, 4].astype(jnp.int32)
    seg = etype * N + dst
    srcT = src.reshape(NS, NC * GC, EC)
    segT = seg.reshape(NS, NC * GC, EC)

    cnt2 = _sc_count(segT)
    aggx = _sc_agg(x, srcT, segT)
    (h0,) = _block_call([x], [aggx], cnt2, params["enc0"],
                        residual_idx=None, epilogue="plain", epi_ws=[])
    aggh = _sc_agg(h0, srcT, segT)
    mu, logvar = _block_call(
        [h0], [aggh], cnt2, params["enc1"], residual_idx=0, epilogue="latent",
        epi_ws=[params["W_mu"], params["b_mu"].reshape(1, -1),
                params["W_lv"], params["b_lv"].reshape(1, -1)])
    aggz = _sc_agg(mu, srcT, segT)
    (d0,) = _block_call([mu, x], [aggz, aggx], cnt2, params["dec0"],
                        residual_idx=None, epilogue="plain", epi_ws=[])
    aggd = _sc_agg(d0, srcT, segT)
    (out,) = _block_call(
        [d0], [aggd], cnt2, params["dec1"], residual_idx=0, epilogue="proj",
        epi_ws=[params["W_out"], params["b_out"].reshape(1, -1)])
    return out, mu, logvar


# speedup vs baseline: 1.3796x; 1.0627x over previous
import functools

import jax
import jax.numpy as jnp
import numpy as np
from jax import lax
from jax.experimental import pallas as pl
from jax.experimental.pallas import tpu as pltpu
from jax.experimental.pallas import tpu_sc as plsc

N = 10000
E = 320000
R = 8
RN = R * N
BN_EPS = 1e-5

NC = 2
NS = 16
NW = NC * NS
DC = 16

EPT = E // NW
EC = 125
GC = EPT // EC
AROWS = RN // NS
ZROWS = 625
FROWS = N // NS

SL = 5

TN = 1000
NT = N // TN

@functools.lru_cache(maxsize=1)
def _mesh():
    return plsc.VectorSubcoreMesh(
        core_axis_name="c", subcore_axis_name="s", num_cores=NC, num_subcores=NS)


def _sc_count(segT):
    @functools.partial(
        pl.kernel,
        out_type=jax.ShapeDtypeStruct((NC, RN, DC), jnp.float32),
        mesh=_mesh(),
        compiler_params=pltpu.CompilerParams(use_tc_tiling_on_sc=False),
        scratch_types=[
            pltpu.VMEM_SHARED((RN, DC), jnp.float32),
            pltpu.VMEM((GC, EC), jnp.int32),
            pltpu.VMEM((EC, DC), jnp.float32),
            pltpu.VMEM((ZROWS, DC), jnp.float32),
        ],
    )
    def k(seg_hbm, out_hbm, cnt_sh, seg_v, ones_v, zbuf):
        c = lax.axis_index("c")
        s = lax.axis_index("s")
        pltpu.sync_copy(seg_hbm.at[s, pl.ds(c * GC, GC)], seg_v)

        def fill_z(i, carry):
            zbuf[i, :] = jnp.zeros((DC,), jnp.float32)
            return carry

        lax.fori_loop(0, ZROWS, fill_z, 0)

        def fill_o(i, carry):
            ones_v[i, :] = jnp.ones((DC,), jnp.float32)
            return carry

        lax.fori_loop(0, EC, fill_o, 0)

        for z in range(AROWS // ZROWS):
            pltpu.sync_copy(zbuf, cnt_sh.at[pl.ds(s * AROWS + z * ZROWS, ZROWS)])
        plsc.subcore_barrier()

        def edge_step(j, carry):
            pltpu.sync_copy(ones_v, cnt_sh.at[seg_v.at[j]], add=True)
            return carry

        lax.fori_loop(0, GC, edge_step, 0)
        plsc.subcore_barrier()
        pltpu.sync_copy(
            cnt_sh.at[pl.ds(s * AROWS, AROWS)],
            out_hbm.at[c, pl.ds(s * AROWS, AROWS), :],
        )

    return k(segT)


def _sc_agg(f, srcT, segT):
    D = f.shape[1]
    C = D // DC
    CH = C // NC

    @functools.partial(
        pl.kernel,
        out_type=jax.ShapeDtypeStruct((RN, D), jnp.float32),
        mesh=_mesh(),
        compiler_params=pltpu.CompilerParams(use_tc_tiling_on_sc=False),
        scratch_types=[
            pltpu.VMEM_SHARED((N, DC), jnp.float32),
            pltpu.VMEM_SHARED((RN, DC), jnp.float32),
            pltpu.VMEM((GC, EC), jnp.int32),
            pltpu.VMEM((GC, EC), jnp.int32),
            [pltpu.VMEM((EC, DC), jnp.float32) for _ in range(SL)],
            pltpu.VMEM((ZROWS, DC), jnp.float32),
            [pltpu.SemaphoreType.DMA for _ in range(SL)],
            [pltpu.SemaphoreType.DMA for _ in range(SL)],
        ],
    )
    def k(f_hbm, src_hbm, seg_hbm, out_hbm,
          fch_sh, agg_sh, src_v, seg_v, gbufs, zbuf, gsems, ssems):
        c = lax.axis_index("c")
        s = lax.axis_index("s")

        def fill_z(i, carry):
            zbuf[i, :] = jnp.zeros((DC,), jnp.float32)
            return carry

        lax.fori_loop(0, ZROWS, fill_z, 0)

        for ci in range(CH):
            col = (c * CH + ci) * DC
            for z in range(AROWS // ZROWS):
                pltpu.sync_copy(zbuf, agg_sh.at[pl.ds(s * AROWS + z * ZROWS, ZROWS)])
            pltpu.sync_copy(
                f_hbm.at[pl.ds(s * FROWS, FROWS), pl.ds(col, DC)],
                fch_sh.at[pl.ds(s * FROWS, FROWS)],
            )
            plsc.subcore_barrier()

            for w in range(NC):
                pltpu.sync_copy(src_hbm.at[s, pl.ds(w * GC, GC)], src_v)
                pltpu.sync_copy(seg_hbm.at[s, pl.ds(w * GC, GC)], seg_v)

                def edge_step(sp, carry):
                    gds = [
                        pltpu.async_copy(
                            fch_sh.at[src_v.at[sp * SL + t]], gbufs[t], gsems[t])
                        for t in range(SL)
                    ]
                    sds = []
                    for t in range(SL):
                        gds[t].wait()
                        sds.append(pltpu.async_copy(
                            gbufs[t], agg_sh.at[seg_v.at[sp * SL + t]],
                            ssems[t], add=True))
                    for t in range(SL):
                        sds[t].wait()
                    return carry

                lax.fori_loop(0, GC // SL, edge_step, 0)
            plsc.subcore_barrier()
            pltpu.sync_copy(
                agg_sh.at[pl.ds(s * AROWS, AROWS)],
                out_hbm.at[pl.ds(s * AROWS, AROWS), pl.ds(col, DC)],
            )

    return k(f, srcT, segT)


def _block_call(feats, aggs, cnt2, bp, *, residual_idx, epilogue, epi_ws):
    P = len(feats)
    dims = [f.shape[1] for f in feats]
    offs = np.concatenate([[0], np.cumsum(dims)]).tolist()
    dout = bp["Wroot"].shape[1]
    inv_bn = float(1.0 / np.sqrt(1.0 + BN_EPS))

    def body(*refs):
        it = iter(refs)
        f_refs = [next(it) for _ in range(P)]
        a_refs = [next(it) for _ in range(P)]
        cnt_ref, wroot, bias, wrel, gamma, beta, pa = (next(it) for _ in range(7))
        e_refs = [next(it) for _ in range(len(epi_ws))]
        if epilogue == "latent":
            o_refs = [next(it), next(it)]
        else:
            o_refs = [next(it)]
        acc = next(it)

        r = pl.program_id(1)

        @pl.when(r == 0)
        def _init():
            a = bias[...]
            for p in range(P):
                a = a + jnp.dot(f_refs[p][...], wroot[offs[p]:offs[p + 1], :],
                                preferred_element_type=jnp.float32)
            acc[...] = a

        @pl.when(r > 0)
        def _accum():
            cnt = cnt_ref[0, :, 0:1] + cnt_ref[1, :, 0:1]
            scale = 1.0 / jnp.maximum(cnt, 1.0)
            a = acc[...]
            for p in range(P):
                m = a_refs[p][...] * scale
                a = a + jnp.dot(m, wrel[0, offs[p]:offs[p + 1], :],
                                preferred_element_type=jnp.float32)
            acc[...] = a

        @pl.when(r == R)
        def _epilogue():
            h = acc[...] * (inv_bn * gamma[...]) + beta[...]
            al = pa[0, 0]
            h = jnp.maximum(h, 0.0) + al * jnp.minimum(h, 0.0)
            if residual_idx is not None:
                h = h + f_refs[residual_idx][...]
            if epilogue == "plain":
                o_refs[0][...] = h
            elif epilogue == "latent":
                o_refs[0][...] = jnp.dot(h, e_refs[0][...],
                                         preferred_element_type=jnp.float32) + e_refs[1][...]
                o_refs[1][...] = jnp.dot(h, e_refs[2][...],
                                         preferred_element_type=jnp.float32) + e_refs[3][...]
            else:
                o_refs[0][...] = jnp.dot(h, e_refs[0][...],
                                         preferred_element_type=jnp.float32) + e_refs[1][...]

    def rm1(r):
        return jnp.maximum(r, 1) - 1

    in_specs = []
    for p in range(P):
        in_specs.append(pl.BlockSpec((TN, dims[p]), lambda i, r: (i, 0)))
    for p in range(P):
        in_specs.append(pl.BlockSpec(
            (TN, dims[p]), lambda i, r: (rm1(r) * NT + i, 0)))
    in_specs.append(pl.BlockSpec((NC, TN, DC), lambda i, r: (0, rm1(r) * NT + i, 0)))
    in_specs.append(pl.BlockSpec((sum(dims), dout), lambda i, r: (0, 0)))
    in_specs.append(pl.BlockSpec((1, dout), lambda i, r: (0, 0)))
    in_specs.append(pl.BlockSpec((1, sum(dims), dout), lambda i, r: (rm1(r), 0, 0)))
    in_specs.append(pl.BlockSpec((1, dout), lambda i, r: (0, 0)))
    in_specs.append(pl.BlockSpec((1, dout), lambda i, r: (0, 0)))
    in_specs.append(pl.BlockSpec((1, 1), lambda i, r: (0, 0)))
    for w in epi_ws:
        in_specs.append(pl.BlockSpec(w.shape, lambda i, r: (0,) * w.ndim))

    if epilogue == "latent":
        lat = epi_ws[0].shape[1]
        out_shape = [jax.ShapeDtypeStruct((N, lat), jnp.float32),
                     jax.ShapeDtypeStruct((N, lat), jnp.float32)]
        out_specs = [pl.BlockSpec((TN, lat), lambda i, r: (i, 0)),
                     pl.BlockSpec((TN, lat), lambda i, r: (i, 0))]
    else:
        od = epi_ws[0].shape[1] if epilogue == "proj" else dout
        out_shape = [jax.ShapeDtypeStruct((N, od), jnp.float32)]
        out_specs = [pl.BlockSpec((TN, od), lambda i, r: (i, 0))]

    wrel3 = bp["Wrel"]
    args = (list(feats) + list(aggs)
            + [cnt2, bp["Wroot"], bp["bias"].reshape(1, dout), wrel3,
               bp["gamma"].reshape(1, dout), bp["beta"].reshape(1, dout),
               bp["prelu_a"].reshape(1, 1)]
            + list(epi_ws))

    outs = pl.pallas_call(
        body,
        grid=(NT, R + 1),
        in_specs=in_specs,
        out_specs=out_specs,
        out_shape=out_shape,
        scratch_shapes=[pltpu.VMEM((TN, dout), jnp.float32)],
    )(*args)
    return outs


def kernel(x, edge_index, edge_attr, params):
    src = edge_index[0]
    dst = edge_index[1]
    etype = edge_attr[:, 4].astype(jnp.int32)
    seg = etype * N + dst
    srcT = src.reshape(NS, NC * GC, EC)
    segT = seg.reshape(NS, NC * GC, EC)

    cnt2 = _sc_count(segT)
    aggx = _sc_agg(x, srcT, segT)
    (h0,) = _block_call([x], [aggx], cnt2, params["enc0"],
                        residual_idx=None, epilogue="plain", epi_ws=[])
    aggh = _sc_agg(h0, srcT, segT)
    mu, logvar = _block_call(
        [h0], [aggh], cnt2, params["enc1"], residual_idx=0, epilogue="latent",
        epi_ws=[params["W_mu"], params["b_mu"].reshape(1, -1),
                params["W_lv"], params["b_lv"].reshape(1, -1)])
    aggz = _sc_agg(mu, srcT, segT)
    (d0,) = _block_call([mu, x], [aggz, aggx], cnt2, params["dec0"],
                        residual_idx=None, epilogue="plain", epi_ws=[])
    aggd = _sc_agg(d0, srcT, segT)
    (out,) = _block_call(
        [d0], [aggd], cnt2, params["dec1"], residual_idx=0, epilogue="proj",
        epi_ws=[params["W_out"], params["b_out"].reshape(1, -1)])
    return out, mu, logvar


# speedup vs baseline: 1.4233x; 1.0317x over previous
import functools

import jax
import jax.numpy as jnp
import numpy as np
from jax import lax
from jax.experimental import pallas as pl
from jax.experimental.pallas import tpu as pltpu
from jax.experimental.pallas import tpu_sc as plsc

N = 10000
E = 320000
R = 8
RN = R * N
BN_EPS = 1e-5

NC = 2
NS = 16
NW = NC * NS
DC = 16

EPT = E // NW
EC = 125
GC = EPT // EC
AROWS = RN // NS
ZROWS = 250
FROWS = N // NS

SL = 8

TN = 1000
NT = N // TN

@functools.lru_cache(maxsize=1)
def _mesh():
    return plsc.VectorSubcoreMesh(
        core_axis_name="c", subcore_axis_name="s", num_cores=NC, num_subcores=NS)


def _sc_count(segT):
    @functools.partial(
        pl.kernel,
        out_type=jax.ShapeDtypeStruct((NC, RN, DC), jnp.float32),
        mesh=_mesh(),
        compiler_params=pltpu.CompilerParams(use_tc_tiling_on_sc=False),
        scratch_types=[
            pltpu.VMEM_SHARED((RN, DC), jnp.float32),
            pltpu.VMEM((GC, EC), jnp.int32),
            pltpu.VMEM((EC, DC), jnp.float32),
            pltpu.VMEM((ZROWS, DC), jnp.float32),
        ],
    )
    def k(seg_hbm, out_hbm, cnt_sh, seg_v, ones_v, zbuf):
        c = lax.axis_index("c")
        s = lax.axis_index("s")
        pltpu.sync_copy(seg_hbm.at[s, pl.ds(c * GC, GC)], seg_v)

        def fill_z(i, carry):
            zbuf[i, :] = jnp.zeros((DC,), jnp.float32)
            return carry

        lax.fori_loop(0, ZROWS, fill_z, 0)

        def fill_o(i, carry):
            ones_v[i, :] = jnp.ones((DC,), jnp.float32)
            return carry

        lax.fori_loop(0, EC, fill_o, 0)

        for z in range(AROWS // ZROWS):
            pltpu.sync_copy(zbuf, cnt_sh.at[pl.ds(s * AROWS + z * ZROWS, ZROWS)])
        plsc.subcore_barrier()

        def edge_step(j, carry):
            pltpu.sync_copy(ones_v, cnt_sh.at[seg_v.at[j]], add=True)
            return carry

        lax.fori_loop(0, GC, edge_step, 0)
        plsc.subcore_barrier()
        pltpu.sync_copy(
            cnt_sh.at[pl.ds(s * AROWS, AROWS)],
            out_hbm.at[c, pl.ds(s * AROWS, AROWS), :],
        )

    return k(segT)


def _sc_agg(f, srcT, segT):
    D = f.shape[1]
    C = D // DC
    CH = C // NC

    @functools.partial(
        pl.kernel,
        out_type=jax.ShapeDtypeStruct((RN, D), jnp.float32),
        mesh=_mesh(),
        compiler_params=pltpu.CompilerParams(use_tc_tiling_on_sc=False),
        scratch_types=[
            pltpu.VMEM_SHARED((N, DC), jnp.float32),
            pltpu.VMEM_SHARED((RN, DC), jnp.float32),
            pltpu.VMEM((GC, EC), jnp.int32),
            pltpu.VMEM((GC, EC), jnp.int32),
            [pltpu.VMEM((EC, DC), jnp.float32) for _ in range(SL)],
            pltpu.VMEM((ZROWS, DC), jnp.float32),
            [pltpu.SemaphoreType.DMA for _ in range(SL)],
            [pltpu.SemaphoreType.DMA for _ in range(SL)],
        ],
    )
    def k(f_hbm, src_hbm, seg_hbm, out_hbm,
          fch_sh, agg_sh, src_v, seg_v, gbufs, zbuf, gsems, ssems):
        c = lax.axis_index("c")
        s = lax.axis_index("s")

        def fill_z(i, carry):
            zbuf[i, :] = jnp.zeros((DC,), jnp.float32)
            return carry

        lax.fori_loop(0, ZROWS, fill_z, 0)

        for ci in range(CH):
            col = (c * CH + ci) * DC
            for z in range(AROWS // ZROWS):
                pltpu.sync_copy(zbuf, agg_sh.at[pl.ds(s * AROWS + z * ZROWS, ZROWS)])
            pltpu.sync_copy(
                f_hbm.at[pl.ds(s * FROWS, FROWS), pl.ds(col, DC)],
                fch_sh.at[pl.ds(s * FROWS, FROWS)],
            )
            plsc.subcore_barrier()

            for w in range(NC):
                pltpu.sync_copy(src_hbm.at[s, pl.ds(w * GC, GC)], src_v)
                pltpu.sync_copy(seg_hbm.at[s, pl.ds(w * GC, GC)], seg_v)

                def edge_step(sp, carry):
                    gds = [
                        pltpu.async_copy(
                            fch_sh.at[src_v.at[sp * SL + t]], gbufs[t], gsems[t])
                        for t in range(SL)
                    ]
                    sds = []
                    for t in range(SL):
                        gds[t].wait()
                        sds.append(pltpu.async_copy(
                            gbufs[t], agg_sh.at[seg_v.at[sp * SL + t]],
                            ssems[t], add=True))
                    for t in range(SL):
                        sds[t].wait()
                    return carry

                lax.fori_loop(0, GC // SL, edge_step, 0)
            plsc.subcore_barrier()
            pltpu.sync_copy(
                agg_sh.at[pl.ds(s * AROWS, AROWS)],
                out_hbm.at[pl.ds(s * AROWS, AROWS), pl.ds(col, DC)],
            )

    return k(f, srcT, segT)


def _block_call(feats, aggs, cnt2, bp, *, residual_idx, epilogue, epi_ws):
    P = len(feats)
    dims = [f.shape[1] for f in feats]
    offs = np.concatenate([[0], np.cumsum(dims)]).tolist()
    dout = bp["Wroot"].shape[1]
    inv_bn = float(1.0 / np.sqrt(1.0 + BN_EPS))

    def body(*refs):
        it = iter(refs)
        f_refs = [next(it) for _ in range(P)]
        a_refs = [next(it) for _ in range(P)]
        cnt_ref, wroot, bias, wrel, gamma, beta, pa = (next(it) for _ in range(7))
        e_refs = [next(it) for _ in range(len(epi_ws))]
        if epilogue == "latent":
            o_refs = [next(it), next(it)]
        else:
            o_refs = [next(it)]
        acc = next(it)

        r = pl.program_id(1)

        @pl.when(r == 0)
        def _init():
            a = bias[...]
            for p in range(P):
                a = a + jnp.dot(f_refs[p][...], wroot[offs[p]:offs[p + 1], :],
                                preferred_element_type=jnp.float32)
            acc[...] = a

        @pl.when(r > 0)
        def _accum():
            cnt = cnt_ref[0, :, 0:1] + cnt_ref[1, :, 0:1]
            scale = 1.0 / jnp.maximum(cnt, 1.0)
            a = acc[...]
            for p in range(P):
                m = a_refs[p][...] * scale
                a = a + jnp.dot(m, wrel[0, offs[p]:offs[p + 1], :],
                                preferred_element_type=jnp.float32)
            acc[...] = a

        @pl.when(r == R)
        def _epilogue():
            h = acc[...] * (inv_bn * gamma[...]) + beta[...]
            al = pa[0, 0]
            h = jnp.maximum(h, 0.0) + al * jnp.minimum(h, 0.0)
            if residual_idx is not None:
                h = h + f_refs[residual_idx][...]
            if epilogue == "plain":
                o_refs[0][...] = h
            elif epilogue == "latent":
                o_refs[0][...] = jnp.dot(h, e_refs[0][...],
                                         preferred_element_type=jnp.float32) + e_refs[1][...]
                o_refs[1][...] = jnp.dot(h, e_refs[2][...],
                                         preferred_element_type=jnp.float32) + e_refs[3][...]
            else:
                o_refs[0][...] = jnp.dot(h, e_refs[0][...],
                                         preferred_element_type=jnp.float32) + e_refs[1][...]

    def rm1(r):
        return jnp.maximum(r, 1) - 1

    in_specs = []
    for p in range(P):
        in_specs.append(pl.BlockSpec((TN, dims[p]), lambda i, r: (i, 0)))
    for p in range(P):
        in_specs.append(pl.BlockSpec(
            (TN, dims[p]), lambda i, r: (rm1(r) * NT + i, 0)))
    in_specs.append(pl.BlockSpec((NC, TN, DC), lambda i, r: (0, rm1(r) * NT + i, 0)))
    in_specs.append(pl.BlockSpec((sum(dims), dout), lambda i, r: (0, 0)))
    in_specs.append(pl.BlockSpec((1, dout), lambda i, r: (0, 0)))
    in_specs.append(pl.BlockSpec((1, sum(dims), dout), lambda i, r: (rm1(r), 0, 0)))
    in_specs.append(pl.BlockSpec((1, dout), lambda i, r: (0, 0)))
    in_specs.append(pl.BlockSpec((1, dout), lambda i, r: (0, 0)))
    in_specs.append(pl.BlockSpec((1, 1), lambda i, r: (0, 0)))
    for w in epi_ws:
        in_specs.append(pl.BlockSpec(w.shape, lambda i, r: (0,) * w.ndim))

    if epilogue == "latent":
        lat = epi_ws[0].shape[1]
        out_shape = [jax.ShapeDtypeStruct((N, lat), jnp.float32),
                     jax.ShapeDtypeStruct((N, lat), jnp.float32)]
        out_specs = [pl.BlockSpec((TN, lat), lambda i, r: (i, 0)),
                     pl.BlockSpec((TN, lat), lambda i, r: (i, 0))]
    else:
        od = epi_ws[0].shape[1] if epilogue == "proj" else dout
        out_shape = [jax.ShapeDtypeStruct((N, od), jnp.float32)]
        out_specs = [pl.BlockSpec((TN, od), lambda i, r: (i, 0))]

    wrel3 = bp["Wrel"]
    args = (list(feats) + list(aggs)
            + [cnt2, bp["Wroot"], bp["bias"].reshape(1, dout), wrel3,
               bp["gamma"].reshape(1, dout), bp["beta"].reshape(1, dout),
               bp["prelu_a"].reshape(1, 1)]
            + list(epi_ws))

    outs = pl.pallas_call(
        body,
        grid=(NT, R + 1),
        in_specs=in_specs,
        out_specs=out_specs,
        out_shape=out_shape,
        scratch_shapes=[pltpu.VMEM((TN, dout), jnp.float32)],
    )(*args)
    return outs


def kernel(x, edge_index, edge_attr, params):
    src = edge_index[0]
    dst = edge_index[1]
    etype = edge_attr[:, 4].astype(jnp.int32)
    seg = etype * N + dst
    srcT = src.reshape(NS, NC * GC, EC)
    segT = seg.reshape(NS, NC * GC, EC)

    cnt2 = _sc_count(segT)
    aggx = _sc_agg(x, srcT, segT)
    (h0,) = _block_call([x], [aggx], cnt2, params["enc0"],
                        residual_idx=None, epilogue="plain", epi_ws=[])
    aggh = _sc_agg(h0, srcT, segT)
    mu, logvar = _block_call(
        [h0], [aggh], cnt2, params["enc1"], residual_idx=0, epilogue="latent",
        epi_ws=[params["W_mu"], params["b_mu"].reshape(1, -1),
                params["W_lv"], params["b_lv"].reshape(1, -1)])
    aggz = _sc_agg(mu, srcT, segT)
    (d0,) = _block_call([mu, x], [aggz, aggx], cnt2, params["dec0"],
                        residual_idx=None, epilogue="plain", epi_ws=[])
    aggd = _sc_agg(d0, srcT, segT)
    (out,) = _block_call(
        [d0], [aggd], cnt2, params["dec1"], residual_idx=0, epilogue="proj",
        epi_ws=[params["W_out"], params["b_out"].reshape(1, -1)])
    return out, mu, logvar
